# Initial kernel scaffold; baseline (speedup 1.0000x reference)
import jax
import jax.numpy as jnp
from jax.experimental import pallas as pl


def kernel(X, Xfield, f, ffield, batch, neighbor, fneighbor, h, alpha, W1, b1, W2, b2, W3, b3, W4, b4):
    raise NotImplementedError("write your pallas kernel here")


# speedup vs baseline: 5.3104x; 5.3104x over previous
import functools

import jax
import jax.numpy as jnp
from jax import lax
from jax.experimental import pallas as pl
from jax.experimental.pallas import tpu as pltpu
from jax.experimental.pallas import tpu_sc as plsc

PI = 3.14159265358
TWO_PI = 2.0 * PI
DR = 1e-05
LN = 16
NW = 32
P_BLK = 128
E_BLK = P_BLK * LN


def _sc_stage_a(xc0, xc1, xc2, nbflat, fneighbor, t3, batch, npad, nq):
    pw = npad // NW
    qw = nq // NW
    mesh = plsc.VectorSubcoreMesh(core_axis_name="c", subcore_axis_name="s")

    @functools.partial(
        pl.kernel, mesh=mesh,
        compiler_params=pltpu.CompilerParams(needs_layout_passes=False, use_tc_tiling_on_sc=False),
        out_type=[
            jax.ShapeDtypeStruct((npad * LN,), jnp.float32),
            jax.ShapeDtypeStruct((nq, LN), jnp.int32),
            jax.ShapeDtypeStruct((nq, LN), jnp.float32),
        ],
        scratch_types=[
            pltpu.VMEM((npad,), jnp.float32),
            pltpu.VMEM((npad,), jnp.float32),
            pltpu.VMEM((npad,), jnp.float32),
            pltpu.VMEM((pw * LN,), jnp.int32),
            pltpu.VMEM((pw * LN,), jnp.float32),
            pltpu.VMEM((qw,), jnp.int32),
            pltpu.VMEM((qw, LN), jnp.int32),
            pltpu.VMEM((qw, LN), jnp.float32),
            pltpu.SemaphoreType.DMA,
        ],
    )
    def k(xc0_h, xc1_h, xc2_h, nb_h, fn_h, t3_h, batch_h,
          s2_h, nbout_h, cout_h,
          x0_v, x1_v, x2_v, nb_v, s2_v, bidx_v, nbr_v, crow_v, sem):
        w = lax.axis_index("s") * 2 + lax.axis_index("c")
        bbase = w * qw
        pltpu.sync_copy(batch_h.at[pl.ds(bbase, qw)], bidx_v)
        pltpu.async_copy(fn_h.at[bidx_v], nbr_v, sem).wait()
        pltpu.sync_copy(nbr_v, nbout_h.at[pl.ds(bbase, qw)])
        pltpu.async_copy(t3_h.at[bidx_v], crow_v, sem).wait()
        pltpu.sync_copy(crow_v, cout_h.at[pl.ds(bbase, qw)])
        pltpu.sync_copy(xc0_h, x0_v)
        pltpu.sync_copy(xc1_h, x1_v)
        pltpu.sync_copy(xc2_h, x2_v)
        pbase = w * pw
        pltpu.sync_copy(nb_h.at[pl.ds(pbase * LN, pw * LN)], nb_v)

        def body(p, carry):
            idxv = nb_v[pl.ds(p * LN, LN)]
            nn = jnp.full((LN,), pbase + p, jnp.int32)
            acc = jnp.zeros((LN,), jnp.float32)
            for xv in (x0_v, x1_v, x2_v):
                g = plsc.load_gather(xv, [idxv])
                c = plsc.load_gather(xv, [nn])
                df = jnp.abs(c - g)
                mm = jnp.minimum(df, TWO_PI - df)
                acc = acc + mm * mm
            s2_v[pl.ds(p * LN, LN)] = acc
            return carry

        lax.fori_loop(0, pw, body, 0)
        pltpu.sync_copy(s2_v, s2_h.at[pl.ds(pbase * LN, pw * LN)])

    return k(xc0, xc1, xc2, nbflat, fneighbor, t3, batch)


def _sc_stage_b(t2, nbflat, nedge):
    ew = nedge // NW
    chunk = 128
    nchunk = ew // chunk
    mesh = plsc.VectorSubcoreMesh(core_axis_name="c", subcore_axis_name="s")

    @functools.partial(
        pl.kernel, mesh=mesh,
        compiler_params=pltpu.CompilerParams(needs_layout_passes=False, use_tc_tiling_on_sc=False),
        out_type=jax.ShapeDtypeStruct((nedge, LN), jnp.float32),
        scratch_types=[
            pltpu.VMEM((chunk,), jnp.int32),
            pltpu.VMEM((chunk, LN), jnp.float32),
            pltpu.SemaphoreType.DMA,
        ],
    )
    def k(t2_h, idx_h, out_h, idx_v, rows_v, sem):
        w = lax.axis_index("s") * 2 + lax.axis_index("c")
        base = w * ew

        def body(kk, carry):
            off = base + kk * chunk
            pltpu.sync_copy(idx_h.at[pl.ds(off, chunk)], idx_v)
            pltpu.async_copy(t2_h.at[idx_v], rows_v, sem).wait()
            pltpu.sync_copy(rows_v, out_h.at[pl.ds(off, chunk)])
            return carry

        lax.fori_loop(0, nchunk, body, 0)

    return k(t2, nbflat)


def _mlp(rcol, w1r, b1r, w2t, b2r, w3t, b3r, w4c, b4s, alpha):
    h1 = jnp.tanh(rcol * w1r + b1r)
    h2 = jnp.tanh(jnp.dot(h1, w2t, preferred_element_type=jnp.float32) + b2r)
    h3 = jnp.tanh(jnp.dot(h2, w3t, preferred_element_type=jnp.float32) + b3r)
    out = jnp.dot(h3, w4c, preferred_element_type=jnp.float32) + b4s
    return out * jax.nn.sigmoid(10.0 * (1.0 - rcol)) * alpha


def _sel_mats(e_blk, p_blk):
    e_over = lax.broadcasted_iota(jnp.int32, (e_blk, p_blk), 0) // LN
    p_io = lax.broadcasted_iota(jnp.int32, (e_blk, p_blk), 1)
    sel_t = (e_over == p_io).astype(jnp.float32)
    p_io2 = lax.broadcasted_iota(jnp.int32, (p_blk, e_blk), 0)
    e_over2 = lax.broadcasted_iota(jnp.int32, (p_blk, e_blk), 1) // LN
    sel = (p_io2 == e_over2).astype(jnp.float32)
    return sel_t, sel


def _tc_a_body(sc_ref, s2_ref, w1r, b1r, w2t, b2r, w3t, b3r, w4c, b4s,
               out_ref):
    s2 = s2_ref[...]
    sel_t, sel = _sel_mats(E_BLK, P_BLK)
    a = jnp.dot(sel_t, s2, preferred_element_type=jnp.float32, precision=lax.Precision.HIGHEST)
    lane = lax.broadcasted_iota(jnp.int32, (E_BLK, LN), 1)
    emod = lax.broadcasted_iota(jnp.int32, (E_BLK, LN), 0) % LN
    s2col = jnp.sum(jnp.where(lane == emod, a, 0.0), axis=1, keepdims=True)
    hval, alpha = sc_ref[0, 0], sc_ref[0, 1]
    r = jnp.sqrt(s2col) / hval
    args = (w1r[...], b1r[...], w2t[...], b2r[...], w3t[...], b3r[...],
            w4c[...], b4s[...], alpha)
    wv = _mlp(r, *args)
    w0 = _mlp(jnp.zeros((P_BLK, 1), jnp.float32), *args)
    out_ref[...] = w0 + jnp.dot(sel, wv, preferred_element_type=jnp.float32, precision=lax.Precision.HIGHEST)


def _tc_b_body(sc_ref, g2_ref, c_ref, w1r, b1r, w2t, b2r, w3t, b3r, w4c,
               b4s, out_ref):
    g = g2_ref[...]
    cp = c_ref[...]
    sel_t, sel = _sel_mats(E_BLK, P_BLK)
    cb = jnp.dot(sel_t, cp, preferred_element_type=jnp.float32, precision=lax.Precision.HIGHEST)
    lane = lax.broadcasted_iota(jnp.int32, (E_BLK, LN), 1)
    xmask = (lane >= 8) & (lane < 11)
    dv = cb - g
    t1 = jnp.abs(dv)
    sgn = -jnp.sign(dv) * jnp.sign(dv + PI) * jnp.sign(dv - PI)
    ov = sgn * jnp.minimum(t1, TWO_PI - t1)
    ov = jnp.where(xmask, ov, 0.0)
    ox = jnp.sum(jnp.where(lane == 8, ov, 0.0), axis=1, keepdims=True)
    oy = jnp.sum(jnp.where(lane == 9, ov, 0.0), axis=1, keepdims=True)
    oz = jnp.sum(jnp.where(lane == 10, ov, 0.0), axis=1, keepdims=True)
    out2 = (ox * ox + oy * oy) + oz * oz
    r0 = jnp.sqrt(out2)
    hval, alpha = sc_ref[0, 0], sc_ref[0, 1]
    dis = r0 / hval
    args = (w1r[...], b1r[...], w2t[...], b2r[...], w3t[...], b3r[...],
            w4c[...], b4s[...], alpha)
    wd = _mlp(dis, *args)
    dwdr = (_mlp(dis + DR, *args) - _mlp(dis - DR, *args)) / (2.0 * DR) / hval
    disv = ov / r0
    rnb = jnp.sum(jnp.where(lane == 11, g, 0.0), axis=1, keepdims=True)
    fn1 = jnp.sum(jnp.where(lane == 1, g, 0.0), axis=1, keepdims=True)
    ffb = jnp.sum(jnp.where(lane == 0, cb, 0.0), axis=1, keepdims=True)
    fmask = lane < 8
    valrho = jnp.where(fmask, g, 0.0) * (wd / rnb)
    vald = ((fn1 - ffb) * dwdr) * disv
    rho_p = jnp.dot(sel, valrho, preferred_element_type=jnp.float32, precision=lax.Precision.HIGHEST)
    d_p = jnp.dot(sel, vald, preferred_element_type=jnp.float32, precision=lax.Precision.HIGHEST)
    w0 = _mlp(jnp.zeros((P_BLK, 1), jnp.float32), *args)
    rho_f = w0 + jnp.dot(sel, wd, preferred_element_type=jnp.float32, precision=lax.Precision.HIGHEST)
    plane = lax.broadcasted_iota(jnp.int32, (P_BLK, LN), 1)
    out_ref[...] = jnp.where(plane < 8, rho_p, d_p / rho_f)


def kernel(X, Xfield, f, ffield, batch, neighbor, fneighbor, h, alpha,
           W1, b1, W2, b2, W3, b3, W4, b4):
    n = X.shape[0]
    nfield = Xfield.shape[0]
    nq = batch.shape[0]
    npad = ((n + NW * 8 - 1) // (NW * 8)) * (NW * 8)
    while npad % P_BLK:
        npad += NW * 8
    padrows = npad - n

    xp = jnp.concatenate([X, jnp.zeros((padrows, 3), jnp.float32)], axis=0)
    nbp = jnp.concatenate(
        [neighbor, jnp.zeros((padrows, LN), jnp.int32)], axis=0)
    t3 = jnp.concatenate([
        ffield[:, None], jnp.zeros((nfield, 7), jnp.float32), Xfield,
        jnp.zeros((nfield, 5), jnp.float32)], axis=1)

    s2flat, nbrows, crows = _sc_stage_a(
        xp[:, 0], xp[:, 1], xp[:, 2], nbp.reshape(-1), fneighbor, t3, batch,
        npad, nq)
    s2 = s2flat.reshape(npad, LN)

    sc = jnp.stack([h.astype(jnp.float32), alpha.astype(jnp.float32)]
                   ).reshape(1, 2)
    w1r = W1.T.reshape(1, 20)
    b1r = b1.reshape(1, 20)
    w2t = W2.T
    b2r = b2.reshape(1, 100)
    w3t = W3.T
    b3r = b3.reshape(1, 20)
    w4c = W4.T.reshape(20, 1)
    b4s = b4.reshape(1, 1)
    wargs = (w1r, b1r, w2t, b2r, w3t, b3r, w4c, b4s)
    wspecs = [pl.BlockSpec(a.shape, lambda g: (0, 0)) for a in wargs]
    hspec = pl.BlockSpec((1, 2), lambda g: (0, 0))

    rho_all = pl.pallas_call(
        _tc_a_body,
        grid=(npad // P_BLK,),
        in_specs=[hspec, pl.BlockSpec((P_BLK, LN), lambda g: (g, 0))] + wspecs,
        out_specs=pl.BlockSpec((P_BLK, 1), lambda g: (g, 0)),
        out_shape=jax.ShapeDtypeStruct((npad, 1), jnp.float32),
    )(sc, s2, *wargs)

    t2 = jnp.concatenate([f, X, rho_all[:n], jnp.zeros((n, 4), jnp.float32)],
                         axis=1)
    g2 = _sc_stage_b(t2, nbrows.reshape(-1), nq * LN)

    out = pl.pallas_call(
        _tc_b_body,
        grid=(nq // P_BLK,),
        in_specs=[hspec,
                  pl.BlockSpec((E_BLK, LN), lambda g: (g, 0)),
                  pl.BlockSpec((P_BLK, LN), lambda g: (g, 0))] + wspecs,
        out_specs=pl.BlockSpec((P_BLK, LN), lambda g: (g, 0)),
        out_shape=jax.ShapeDtypeStruct((nq, LN), jnp.float32),
    )(sc, g2, crows, *wargs)

    return out[:, :8], out[:, 8:11]


# speedup vs baseline: 10.7037x; 2.0156x over previous
import functools

import jax
import jax.numpy as jnp
from jax import lax
from jax.experimental import pallas as pl
from jax.experimental.pallas import tpu as pltpu
from jax.experimental.pallas import tpu_sc as plsc

PI = 3.14159265358
TWO_PI = 2.0 * PI
DR = 1e-05
LN = 16
NW = 32
P_BLK = 256
E_BLK = P_BLK * LN


def _sc_stage_a(xc0, xc1, xc2, nbflat, fneighbor, t3, batch, npad, nq):
    pw = npad // NW
    qw = nq // NW
    mesh = plsc.VectorSubcoreMesh(core_axis_name="c", subcore_axis_name="s")

    @functools.partial(
        pl.kernel, mesh=mesh,
        compiler_params=pltpu.CompilerParams(needs_layout_passes=False, use_tc_tiling_on_sc=False),
        out_type=[
            jax.ShapeDtypeStruct((npad * LN,), jnp.float32),
            jax.ShapeDtypeStruct((nq, LN), jnp.int32),
            jax.ShapeDtypeStruct((nq, LN), jnp.float32),
        ],
        scratch_types=[
            pltpu.VMEM((npad,), jnp.float32),
            pltpu.VMEM((npad,), jnp.float32),
            pltpu.VMEM((npad,), jnp.float32),
            pltpu.VMEM((pw * LN,), jnp.int32),
            pltpu.VMEM((pw * LN,), jnp.float32),
            pltpu.VMEM((qw,), jnp.int32),
            pltpu.VMEM((qw, LN), jnp.int32),
            pltpu.VMEM((qw, LN), jnp.float32),
            pltpu.SemaphoreType.DMA,
        ],
    )
    def k(xc0_h, xc1_h, xc2_h, nb_h, fn_h, t3_h, batch_h,
          s2_h, nbout_h, cout_h,
          x0_v, x1_v, x2_v, nb_v, s2_v, bidx_v, nbr_v, crow_v, sem):
        w = lax.axis_index("s") * 2 + lax.axis_index("c")
        bbase = w * qw
        pltpu.sync_copy(batch_h.at[pl.ds(bbase, qw)], bidx_v)
        pltpu.async_copy(fn_h.at[bidx_v], nbr_v, sem).wait()
        pltpu.sync_copy(nbr_v, nbout_h.at[pl.ds(bbase, qw)])
        pltpu.async_copy(t3_h.at[bidx_v], crow_v, sem).wait()
        pltpu.sync_copy(crow_v, cout_h.at[pl.ds(bbase, qw)])
        pltpu.sync_copy(xc0_h, x0_v)
        pltpu.sync_copy(xc1_h, x1_v)
        pltpu.sync_copy(xc2_h, x2_v)
        pbase = w * pw
        pltpu.sync_copy(nb_h.at[pl.ds(pbase * LN, pw * LN)], nb_v)

        def body(p, carry):
            idxv = nb_v[pl.ds(p * LN, LN)]
            nn = jnp.full((LN,), pbase + p, jnp.int32)
            acc = jnp.zeros((LN,), jnp.float32)
            for xv in (x0_v, x1_v, x2_v):
                g = plsc.load_gather(xv, [idxv])
                c = plsc.load_gather(xv, [nn])
                df = jnp.abs(c - g)
                mm = jnp.minimum(df, TWO_PI - df)
                acc = acc + mm * mm
            s2_v[pl.ds(p * LN, LN)] = acc
            return carry

        lax.fori_loop(0, pw, body, 0)
        pltpu.sync_copy(s2_v, s2_h.at[pl.ds(pbase * LN, pw * LN)])

    return k(xc0, xc1, xc2, nbflat, fneighbor, t3, batch)


def _sc_stage_b(t2, nbflat, nedge):
    ew = nedge // NW
    chunk = 128
    nchunk = ew // chunk
    mesh = plsc.VectorSubcoreMesh(core_axis_name="c", subcore_axis_name="s")

    @functools.partial(
        pl.kernel, mesh=mesh,
        compiler_params=pltpu.CompilerParams(needs_layout_passes=False, use_tc_tiling_on_sc=False),
        out_type=jax.ShapeDtypeStruct((nedge, LN), jnp.float32),
        scratch_types=[
            pltpu.VMEM((chunk,), jnp.int32),
            pltpu.VMEM((chunk, LN), jnp.float32),
            pltpu.SemaphoreType.DMA,
        ],
    )
    def k(t2_h, idx_h, out_h, idx_v, rows_v, sem):
        w = lax.axis_index("s") * 2 + lax.axis_index("c")
        base = w * ew

        def body(kk, carry):
            off = base + kk * chunk
            pltpu.sync_copy(idx_h.at[pl.ds(off, chunk)], idx_v)
            pltpu.async_copy(t2_h.at[idx_v], rows_v, sem).wait()
            pltpu.sync_copy(rows_v, out_h.at[pl.ds(off, chunk)])
            return carry

        lax.fori_loop(0, nchunk, body, 0)

    return k(t2, nbflat)


def _mlp(rcol, w1r, b1r, w2t, b2r, w3t, b3r, w4c, b4s, alpha):
    h1 = jnp.tanh(rcol * w1r + b1r)
    h2 = jnp.tanh(jnp.dot(h1, w2t, preferred_element_type=jnp.float32) + b2r)
    h3 = jnp.tanh(jnp.dot(h2, w3t, preferred_element_type=jnp.float32) + b3r)
    out = jnp.dot(h3, w4c, preferred_element_type=jnp.float32) + b4s
    return out * jax.nn.sigmoid(10.0 * (1.0 - rcol)) * alpha


def _expand(x, k=None):
    p, ln = x.shape
    return jnp.broadcast_to(x[:, None, :], (p, LN, ln)).reshape(p * LN, ln)


def _reduce16(x):
    e, kk = x.shape
    return jnp.sum(x.reshape(e // LN, LN, kk), axis=1)


def _tc_a_body(sc_ref, s2_ref, w1r, b1r, w2t, b2r, w3t, b3r, w4c, b4s,
               out_ref):
    s2 = s2_ref[...]
    a = _expand(s2)
    lane = lax.broadcasted_iota(jnp.int32, (E_BLK, LN), 1)
    emod = lax.broadcasted_iota(jnp.int32, (E_BLK, LN), 0) % LN
    s2col = jnp.sum(jnp.where(lane == emod, a, 0.0), axis=1, keepdims=True)
    hval, alpha = sc_ref[0, 0], sc_ref[0, 1]
    r = jnp.sqrt(s2col) / hval
    args = (w1r[...], b1r[...], w2t[...], b2r[...], w3t[...], b3r[...],
            w4c[...], b4s[...], alpha)
    wv = _mlp(r, *args)
    w0 = _mlp(jnp.zeros((P_BLK, 1), jnp.float32), *args)
    out_ref[...] = w0 + _reduce16(wv)


def _tc_b_body(sc_ref, g2_ref, c_ref, w1r, b1r, w2t, b2r, w3t, b3r, w4c,
               b4s, out_ref):
    g = g2_ref[...]
    cp = c_ref[...]
    cb = _expand(cp)
    lane = lax.broadcasted_iota(jnp.int32, (E_BLK, LN), 1)
    xmask = (lane >= 8) & (lane < 11)
    dv = cb - g
    t1 = jnp.abs(dv)
    sgn_nz = jnp.where((dv > PI) | ((dv < 0.0) & (dv > -PI)), -1.0, 1.0)
    sgn = jnp.where((t1 == PI) | (dv == 0.0), 0.0, sgn_nz)
    ov = sgn * jnp.minimum(t1, TWO_PI - t1)
    ov = jnp.where(xmask, ov, 0.0)
    ox = jnp.sum(jnp.where(lane == 8, ov, 0.0), axis=1, keepdims=True)
    oy = jnp.sum(jnp.where(lane == 9, ov, 0.0), axis=1, keepdims=True)
    oz = jnp.sum(jnp.where(lane == 10, ov, 0.0), axis=1, keepdims=True)
    out2 = (ox * ox + oy * oy) + oz * oz
    r0 = jnp.sqrt(out2)
    hval, alpha = sc_ref[0, 0], sc_ref[0, 1]
    dis = r0 / hval
    args = (w1r[...], b1r[...], w2t[...], b2r[...], w3t[...], b3r[...],
            w4c[...], b4s[...], alpha)
    wd = _mlp(dis, *args)
    dwdr = (_mlp(dis + DR, *args) - _mlp(dis - DR, *args)) / (2.0 * DR) / hval
    disv = ov * (1.0 / r0)
    rnb = jnp.sum(jnp.where(lane == 11, g, 0.0), axis=1, keepdims=True)
    fn1 = jnp.sum(jnp.where(lane == 1, g, 0.0), axis=1, keepdims=True)
    ffb = jnp.sum(jnp.where(lane == 0, cb, 0.0), axis=1, keepdims=True)
    fmask = lane < 8
    valrho = jnp.where(fmask, g, 0.0) * (wd / rnb)
    vald = ((fn1 - ffb) * dwdr) * disv
    rho_p = _reduce16(valrho)
    d_p = _reduce16(vald)
    w0 = _mlp(jnp.zeros((P_BLK, 1), jnp.float32), *args)
    rho_f = w0 + _reduce16(wd)
    plane = lax.broadcasted_iota(jnp.int32, (P_BLK, LN), 1)
    out_ref[...] = jnp.where(plane < 8, rho_p, d_p / rho_f)


def kernel(X, Xfield, f, ffield, batch, neighbor, fneighbor, h, alpha,
           W1, b1, W2, b2, W3, b3, W4, b4):
    n = X.shape[0]
    nfield = Xfield.shape[0]
    nq = batch.shape[0]
    npad = ((n + NW * 8 - 1) // (NW * 8)) * (NW * 8)
    while npad % P_BLK:
        npad += NW * 8
    padrows = npad - n

    xp = jnp.concatenate([X, jnp.zeros((padrows, 3), jnp.float32)], axis=0)
    nbp = jnp.concatenate(
        [neighbor, jnp.zeros((padrows, LN), jnp.int32)], axis=0)
    t3 = jnp.concatenate([
        ffield[:, None], jnp.zeros((nfield, 7), jnp.float32), Xfield,
        jnp.zeros((nfield, 5), jnp.float32)], axis=1)

    s2flat, nbrows, crows = _sc_stage_a(
        xp[:, 0], xp[:, 1], xp[:, 2], nbp.reshape(-1), fneighbor, t3, batch,
        npad, nq)
    s2 = s2flat.reshape(npad, LN)

    sc = jnp.stack([h.astype(jnp.float32), alpha.astype(jnp.float32)]
                   ).reshape(1, 2)
    w1r = W1.T.reshape(1, 20)
    b1r = b1.reshape(1, 20)
    w2t = W2.T
    b2r = b2.reshape(1, 100)
    w3t = W3.T
    b3r = b3.reshape(1, 20)
    w4c = W4.T.reshape(20, 1)
    b4s = b4.reshape(1, 1)
    wargs = (w1r, b1r, w2t, b2r, w3t, b3r, w4c, b4s)
    wspecs = [pl.BlockSpec(a.shape, lambda g: (0, 0)) for a in wargs]
    hspec = pl.BlockSpec((1, 2), lambda g: (0, 0))

    rho_all = pl.pallas_call(
        _tc_a_body,
        grid=(npad // P_BLK,),
        in_specs=[hspec, pl.BlockSpec((P_BLK, LN), lambda g: (g, 0))] + wspecs,
        out_specs=pl.BlockSpec((P_BLK, 1), lambda g: (g, 0)),
        out_shape=jax.ShapeDtypeStruct((npad, 1), jnp.float32),
    )(sc, s2, *wargs)

    t2 = jnp.concatenate([f, X, rho_all[:n], jnp.zeros((n, 4), jnp.float32)],
                         axis=1)
    g2 = _sc_stage_b(t2, nbrows.reshape(-1), nq * LN)

    out = pl.pallas_call(
        _tc_b_body,
        grid=(nq // P_BLK,),
        in_specs=[hspec,
                  pl.BlockSpec((E_BLK, LN), lambda g: (g, 0)),
                  pl.BlockSpec((P_BLK, LN), lambda g: (g, 0))] + wspecs,
        out_specs=pl.BlockSpec((P_BLK, LN), lambda g: (g, 0)),
        out_shape=jax.ShapeDtypeStruct((nq, LN), jnp.float32),
    )(sc, g2, crows, *wargs)

    return out[:, :8], out[:, 8:11]
